# baseline (speedup 1.0000x reference)
import functools

import jax
import jax.numpy as jnp
from jax import lax
from jax.experimental import pallas as pl
from jax.experimental.pallas import tpu as pltpu
from jax.experimental.pallas import tpu_sc as plsc

B = 64
C = 3
H = W = 128
HW = H * W
CHW = C * HW
P_NUM = 16
NPTS = 1024
NS = 100
E_TOT = P_NUM * NS

_NC = 2
_NW = 32
_BPW = B // _NW


def _sc_gather_body(t_hbm, o_hbm, mr_hbm, tg_hbm, og_hbm,
                    t_v, o_v, mr0_v, mr1_v, idx0_v, idx1_v,
                    tgo0_v, ogo0_v, tgo1_v, ogo1_v,
                    sem_t, sem_o, sem_to, sem_oo):
    cid = lax.axis_index("c")
    sid = lax.axis_index("s")
    wid = sid * _NC + cid
    b0 = wid * _BPW
    b1 = b0 + 1
    lane = lax.iota(jnp.int32, 16)

    def build_idx(mr_v, idx_v):
        def _b(j, _):
            ii = j * 32 + 2 * lane
            r = plsc.load_gather(mr_v, [ii])
            cc = plsc.load_gather(mr_v, [ii + 1])
            idx_v[pl.ds(j * 16, 16)] = r * W + cc
            return 0

        lax.fori_loop(0, NPTS // 16, _b, 0, unroll=4)

    def gather_into(src_v, dst_v, idx_v):
        for ch in range(C):
            def _g(j, _):
                iv = idx_v[pl.ds(j * 16, 16)]
                dst_v[ch, pl.ds(j * 16, 16)] = plsc.load_gather(
                    src_v, [iv + ch * HW])
                return 0

            lax.fori_loop(0, NPTS // 16, _g, 0, unroll=4)

    cp_t0 = pltpu.make_async_copy(t_hbm.at[b0], t_v, sem_t)
    cp_t0.start()
    cp_o0 = pltpu.make_async_copy(o_hbm.at[b0], o_v, sem_o)
    cp_o0.start()
    pltpu.sync_copy(mr_hbm.at[b0], mr0_v)
    pltpu.sync_copy(mr_hbm.at[b1], mr1_v)
    build_idx(mr0_v, idx0_v)
    build_idx(mr1_v, idx1_v)

    cp_t0.wait()
    gather_into(t_v, tgo0_v, idx0_v)
    cp_t1 = pltpu.make_async_copy(t_hbm.at[b1], t_v, sem_t)
    cp_t1.start()
    out_t0 = pltpu.make_async_copy(tgo0_v, tg_hbm.at[b0], sem_to)
    out_t0.start()

    cp_o0.wait()
    gather_into(o_v, ogo0_v, idx0_v)
    cp_o1 = pltpu.make_async_copy(o_hbm.at[b1], o_v, sem_o)
    cp_o1.start()
    out_o0 = pltpu.make_async_copy(ogo0_v, og_hbm.at[b0], sem_oo)
    out_o0.start()

    cp_t1.wait()
    gather_into(t_v, tgo1_v, idx1_v)
    out_t1 = pltpu.make_async_copy(tgo1_v, tg_hbm.at[b1], sem_to)
    out_t1.start()

    cp_o1.wait()
    gather_into(o_v, ogo1_v, idx1_v)
    out_o1 = pltpu.make_async_copy(ogo1_v, og_hbm.at[b1], sem_oo)
    out_o1.start()

    out_t0.wait()
    out_o0.wait()
    out_t1.wait()
    out_o1.wait()


@functools.lru_cache(maxsize=1)
def _get_sc_gather():
    return functools.partial(
        pl.kernel,
        out_type=(jax.ShapeDtypeStruct((B, C, NPTS), jnp.float32),
                  jax.ShapeDtypeStruct((B, C, NPTS), jnp.float32)),
        mesh=plsc.VectorSubcoreMesh(core_axis_name="c", subcore_axis_name="s"),
        compiler_params=pltpu.CompilerParams(needs_layout_passes=False),
        scratch_types=[
            pltpu.VMEM((CHW,), jnp.float32),
            pltpu.VMEM((CHW,), jnp.float32),
            pltpu.VMEM((2 * NPTS,), jnp.int32),
            pltpu.VMEM((2 * NPTS,), jnp.int32),
            pltpu.VMEM((NPTS,), jnp.int32),
            pltpu.VMEM((NPTS,), jnp.int32),
            pltpu.VMEM((C, NPTS), jnp.float32),
            pltpu.VMEM((C, NPTS), jnp.float32),
            pltpu.VMEM((C, NPTS), jnp.float32),
            pltpu.VMEM((C, NPTS), jnp.float32),
            pltpu.SemaphoreType.DMA,
            pltpu.SemaphoreType.DMA,
            pltpu.SemaphoreType.DMA,
            pltpu.SemaphoreType.DMA,
        ],
    )(_sc_gather_body)


def _coefs_body(t_ref, o_ref, s_ref, c_ref):
    t = t_ref[...].reshape(-1, W)
    o = o_ref[...].reshape(-1, W)
    m = t != 0.0
    d = jnp.abs(jnp.where(m, o, 0.0) - t)

    @pl.when(pl.program_id(0) == 0)
    def _init():
        s_ref[0, 0] = 0.0
        c_ref[0, 0] = 0.0

    s_ref[0, 0] += jnp.sum(d)
    c_ref[0, 0] += jnp.sum(m.astype(jnp.float32))


_N_COEFS_BLKS = 8
_COEFS_BB = B // _N_COEFS_BLKS

_coefs_call = pl.pallas_call(
    _coefs_body,
    grid=(_N_COEFS_BLKS,),
    in_specs=[pl.BlockSpec((_COEFS_BB, C, H, W), lambda i: (i, 0, 0, 0)),
              pl.BlockSpec((_COEFS_BB, C, H, W), lambda i: (i, 0, 0, 0))],
    out_specs=[pl.BlockSpec(memory_space=pltpu.SMEM),
               pl.BlockSpec(memory_space=pltpu.SMEM)],
    out_shape=[jax.ShapeDtypeStruct((1, 1), jnp.float32),
               jax.ShapeDtypeStruct((1, 1), jnp.float32)],
)


def _sol_body(tg_ref, og_ref, bas_ref, se_ref, re_ref):
    tg = tg_ref[...].reshape(B * C, NPTS)
    og = og_ref[...].reshape(B * C, NPTS)
    og = jnp.where(tg != 0.0, og, 0.0)
    both = jnp.concatenate([tg, og], axis=0)
    s_abs = jnp.float32(0.0)
    s_rel = jnp.float32(0.0)
    for k in range(P_NUM):
        blk = both[:, k * 64:(k + 1) * 64]
        sol = lax.dot_general(
            blk, bas_ref[k],
            dimension_numbers=(((1,), (1,)), ((), ())),
            preferred_element_type=jnp.float32,
            precision=lax.Precision.HIGHEST)
        ts = sol[:B * C]
        ps = sol[B * C:]
        err = ts - ps
        mask = jnp.abs(ts) > 1e-6
        denom = jnp.where(mask, ts, 1.0)
        rel = jnp.where(mask, err / denom, 0.0)
        s_abs += jnp.sum(jnp.abs(err))
        s_rel += jnp.sum(jnp.abs(rel))
    se_ref[0, 0] = s_abs
    re_ref[0, 0] = s_rel


_sol_call = pl.pallas_call(
    _sol_body,
    out_specs=[pl.BlockSpec(memory_space=pltpu.SMEM),
               pl.BlockSpec(memory_space=pltpu.SMEM)],
    out_shape=[jax.ShapeDtypeStruct((1, 1), jnp.float32),
               jax.ShapeDtypeStruct((1, 1), jnp.float32)],
)


def kernel(mapRecord, targets, outputs, patchIndex, basis):
    del patchIndex

    tg, og = _get_sc_gather()(targets.reshape(B, CHW), outputs.reshape(B, CHW),
                              mapRecord.reshape(B, 2 * NPTS))

    s_sum, cnt = _coefs_call(targets, outputs)

    bas = basis.reshape(P_NUM, NS, 64)
    se, re = _sol_call(tg, og, bas)

    coefs_error = s_sum[0, 0] / cnt[0, 0]
    denom = jnp.float32(B * C * E_TOT)
    return (coefs_error, se[0, 0] / denom, re[0, 0] / denom)


# speedup vs baseline: 1.6494x; 1.6494x over previous
import functools

import jax
import jax.numpy as jnp
from jax import lax
from jax.experimental import pallas as pl
from jax.experimental.pallas import tpu as pltpu
from jax.experimental.pallas import tpu_sc as plsc

B = 64
C = 3
H = W = 128
HW = H * W
CHW = C * HW
P_NUM = 16
NPTS = 1024
NS = 100
E_TOT = P_NUM * NS

_NC = 2
_NW = 32
_BPW = B // _NW


def _sc_gather_body(t_hbm, o_hbm, mr_hbm, tg_hbm, og_hbm,
                    t_v, o_v, mr0_v, mr1_v,
                    tgo0_v, ogo0_v, tgo1_v, ogo1_v,
                    sem_t, sem_o, sem_to, sem_oo):
    cid = lax.axis_index("c")
    sid = lax.axis_index("s")
    wid = sid * _NC + cid
    b0 = wid * _BPW
    b1 = b0 + 1
    lane = lax.iota(jnp.int32, 16)

    cvecs = [jnp.full((16,), ch, jnp.int32) for ch in range(C)]

    def gather_into(src_v, dst_v, mr_v):
        def _g(j, _):
            ii = j * 32 + 2 * lane
            r = plsc.load_gather(mr_v, [ii])
            cc = plsc.load_gather(mr_v, [ii + 1])
            for ch in range(C):
                dst_v[ch, pl.ds(j * 16, 16)] = plsc.load_gather(
                    src_v, [cvecs[ch], r, cc])
            return 0

        lax.fori_loop(0, NPTS // 16, _g, 0, unroll=4)

    cp_t0 = pltpu.make_async_copy(t_hbm.at[b0], t_v, sem_t)
    cp_t0.start()
    cp_o0 = pltpu.make_async_copy(o_hbm.at[b0], o_v, sem_o)
    cp_o0.start()
    pltpu.sync_copy(mr_hbm.at[b0], mr0_v)
    pltpu.sync_copy(mr_hbm.at[b1], mr1_v)

    cp_t0.wait()
    gather_into(t_v, tgo0_v, mr0_v)
    cp_t1 = pltpu.make_async_copy(t_hbm.at[b1], t_v, sem_t)
    cp_t1.start()
    out_t0 = pltpu.make_async_copy(tgo0_v, tg_hbm.at[b0], sem_to)
    out_t0.start()

    cp_o0.wait()
    gather_into(o_v, ogo0_v, mr0_v)
    cp_o1 = pltpu.make_async_copy(o_hbm.at[b1], o_v, sem_o)
    cp_o1.start()
    out_o0 = pltpu.make_async_copy(ogo0_v, og_hbm.at[b0], sem_oo)
    out_o0.start()

    cp_t1.wait()
    gather_into(t_v, tgo1_v, mr1_v)
    out_t1 = pltpu.make_async_copy(tgo1_v, tg_hbm.at[b1], sem_to)
    out_t1.start()

    cp_o1.wait()
    gather_into(o_v, ogo1_v, mr1_v)
    out_o1 = pltpu.make_async_copy(ogo1_v, og_hbm.at[b1], sem_oo)
    out_o1.start()

    out_t0.wait()
    out_o0.wait()
    out_t1.wait()
    out_o1.wait()


@functools.lru_cache(maxsize=1)
def _get_sc_gather():
    return functools.partial(
        pl.kernel,
        out_type=(jax.ShapeDtypeStruct((B, C, NPTS), jnp.float32),
                  jax.ShapeDtypeStruct((B, C, NPTS), jnp.float32)),
        mesh=plsc.VectorSubcoreMesh(core_axis_name="c", subcore_axis_name="s"),
        compiler_params=pltpu.CompilerParams(needs_layout_passes=False),
        scratch_types=[
            pltpu.VMEM((C, H, W), jnp.float32),
            pltpu.VMEM((C, H, W), jnp.float32),
            pltpu.VMEM((2 * NPTS,), jnp.int32),
            pltpu.VMEM((2 * NPTS,), jnp.int32),
            pltpu.VMEM((C, NPTS), jnp.float32),
            pltpu.VMEM((C, NPTS), jnp.float32),
            pltpu.VMEM((C, NPTS), jnp.float32),
            pltpu.VMEM((C, NPTS), jnp.float32),
            pltpu.SemaphoreType.DMA,
            pltpu.SemaphoreType.DMA,
            pltpu.SemaphoreType.DMA,
            pltpu.SemaphoreType.DMA,
        ],
    )(_sc_gather_body)


def _coefs_body(t_ref, o_ref, s_ref, c_ref):
    t = t_ref[...].reshape(-1, W)
    o = o_ref[...].reshape(-1, W)
    m = t != 0.0
    d = jnp.abs(jnp.where(m, o, 0.0) - t)

    @pl.when(pl.program_id(0) == 0)
    def _init():
        s_ref[0, 0] = 0.0
        c_ref[0, 0] = 0.0

    s_ref[0, 0] += jnp.sum(d)
    c_ref[0, 0] += jnp.sum(m.astype(jnp.float32))


_N_COEFS_BLKS = 8
_COEFS_BB = B // _N_COEFS_BLKS

_coefs_call = pl.pallas_call(
    _coefs_body,
    grid=(_N_COEFS_BLKS,),
    in_specs=[pl.BlockSpec((_COEFS_BB, C, H, W), lambda i: (i, 0, 0, 0)),
              pl.BlockSpec((_COEFS_BB, C, H, W), lambda i: (i, 0, 0, 0))],
    out_specs=[pl.BlockSpec(memory_space=pltpu.SMEM),
               pl.BlockSpec(memory_space=pltpu.SMEM)],
    out_shape=[jax.ShapeDtypeStruct((1, 1), jnp.float32),
               jax.ShapeDtypeStruct((1, 1), jnp.float32)],
)


def _sol_body(tg_ref, og_ref, bas_ref, se_ref, re_ref):
    tg = tg_ref[...].reshape(B * C, NPTS)
    og = og_ref[...].reshape(B * C, NPTS)
    og = jnp.where(tg != 0.0, og, 0.0)
    both = jnp.concatenate([tg, og], axis=0)
    s_abs = jnp.float32(0.0)
    s_rel = jnp.float32(0.0)
    for k in range(P_NUM):
        blk = both[:, k * 64:(k + 1) * 64]
        sol = lax.dot_general(
            blk, bas_ref[k],
            dimension_numbers=(((1,), (1,)), ((), ())),
            preferred_element_type=jnp.float32,
            precision=lax.Precision.HIGHEST)
        ts = sol[:B * C]
        ps = sol[B * C:]
        err = ts - ps
        mask = jnp.abs(ts) > 1e-6
        denom = jnp.where(mask, ts, 1.0)
        rel = jnp.where(mask, err / denom, 0.0)
        s_abs += jnp.sum(jnp.abs(err))
        s_rel += jnp.sum(jnp.abs(rel))
    se_ref[0, 0] = s_abs
    re_ref[0, 0] = s_rel


_sol_call = pl.pallas_call(
    _sol_body,
    out_specs=[pl.BlockSpec(memory_space=pltpu.SMEM),
               pl.BlockSpec(memory_space=pltpu.SMEM)],
    out_shape=[jax.ShapeDtypeStruct((1, 1), jnp.float32),
               jax.ShapeDtypeStruct((1, 1), jnp.float32)],
)


def kernel(mapRecord, targets, outputs, patchIndex, basis):
    del patchIndex

    tg, og = _get_sc_gather()(targets, outputs, mapRecord.reshape(B, 2 * NPTS))

    s_sum, cnt = _coefs_call(targets, outputs)

    bas = basis.reshape(P_NUM, NS, 64)
    se, re = _sol_call(tg, og, bas)

    coefs_error = s_sum[0, 0] / cnt[0, 0]
    denom = jnp.float32(B * C * E_TOT)
    return (coefs_error, se[0, 0] / denom, re[0, 0] / denom)
